# baseline (speedup 1.0000x reference)
import functools

import jax
import jax.numpy as jnp
from jax import lax
from jax.experimental import pallas as pl
from jax.experimental.pallas import tpu as pltpu
from jax.experimental.pallas import tpu_sc as plsc

D_PROJ = 128
CUT0 = 20000
CUT1 = 100000
CUT2 = 1000000
V0, D0 = 20000, 128
V1, D1 = 80000, 32
V2, D2 = 900000, 8

N_TOKENS = 4096 * 50
NC, NS = 2, 16
NW = NC * NS
TOK_PER_W = N_TOKENS // NW
CHUNK = 128
CHUNKS_PER_W = TOK_PER_W // CHUNK


def _sc_gather_body(idx_hbm, emb0, emb1, emb2, g0_hbm, g1_hbm, g2_hbm,
                    idx_v, i0_v, i1_v, i2_v, g0_v, g1_v, g2_v, s0, s1, s2):
    wid = lax.axis_index("s") * NC + lax.axis_index("c")

    def chunk_body(ci, carry):
        base = (wid * CHUNKS_PER_W + ci) * CHUNK
        pltpu.sync_copy(idx_hbm.at[pl.ds(base, CHUNK)], idx_v)

        def vec_body(j, c):
            v = idx_v[pl.ds(j * 16, 16)]
            i0_v[pl.ds(j * 16, 16)] = jnp.minimum(v, V0 - 1)
            i1_v[pl.ds(j * 16, 16)] = jnp.clip(v - CUT0, 0, V1 - 1)
            i2_v[pl.ds(j * 16, 16)] = jnp.clip(v - CUT1, 0, V2 - 1)
            return c

        lax.fori_loop(0, CHUNK // 16, vec_body, 0)

        c0 = pltpu.async_copy(emb0.at[i0_v], g0_v, s0)
        c1 = pltpu.async_copy(emb1.at[i1_v], g1_v, s1)
        c2 = pltpu.async_copy(emb2.at[i2_v], g2_v, s2)
        c0.wait()
        c1.wait()
        c2.wait()
        pltpu.sync_copy(g0_v, g0_hbm.at[pl.ds(base, CHUNK)])
        pltpu.sync_copy(g1_v, g1_hbm.at[pl.ds(base, CHUNK)])
        pltpu.sync_copy(g2_v, g2_hbm.at[pl.ds(base, CHUNK)])
        return carry

    lax.fori_loop(0, CHUNKS_PER_W, chunk_body, 0)


@functools.cache
def _sc_gather_kernel():
    return functools.partial(
        pl.kernel,
        mesh=plsc.VectorSubcoreMesh(core_axis_name="c", subcore_axis_name="s"),
        compiler_params=pltpu.CompilerParams(use_tc_tiling_on_sc=False),
        out_type=[
            jax.ShapeDtypeStruct((N_TOKENS, D0), jnp.float32),
            jax.ShapeDtypeStruct((N_TOKENS, D1), jnp.float32),
            jax.ShapeDtypeStruct((N_TOKENS, D2), jnp.float32),
        ],
        scratch_types=[
            pltpu.VMEM((CHUNK,), jnp.int32),
            pltpu.VMEM((CHUNK,), jnp.int32),
            pltpu.VMEM((CHUNK,), jnp.int32),
            pltpu.VMEM((CHUNK,), jnp.int32),
            pltpu.VMEM((CHUNK, D0), jnp.float32),
            pltpu.VMEM((CHUNK, D1), jnp.float32),
            pltpu.VMEM((CHUNK, D2), jnp.float32),
            pltpu.SemaphoreType.DMA,
            pltpu.SemaphoreType.DMA,
            pltpu.SemaphoreType.DMA,
        ],
    )(_sc_gather_body)


BT = 1024
N_BLOCKS = N_TOKENS // BT


def _tc_project_body(idx_ref, g0_ref, g1_ref, g2_ref, p0_ref, p1_ref, p2_ref,
                     out_ref):
    idx = idx_ref[...]
    m0 = (idx < CUT0).astype(jnp.float32)
    m1 = ((idx >= CUT0) & (idx < CUT1)).astype(jnp.float32)
    m2 = (idx >= CUT1).astype(jnp.float32)
    acc = jnp.dot(g0_ref[...] * m0, p0_ref[...],
                  preferred_element_type=jnp.float32)
    acc += jnp.dot(g1_ref[...] * m1, p1_ref[...],
                   preferred_element_type=jnp.float32)
    acc += jnp.dot(g2_ref[...] * m2, p2_ref[...],
                   preferred_element_type=jnp.float32)
    out_ref[...] = acc


def _tc_project(idx2d, g0, g1, g2, p0t, p1t, p2t):
    return pl.pallas_call(
        _tc_project_body,
        grid=(N_BLOCKS,),
        in_specs=[
            pl.BlockSpec((BT, 1), lambda i: (i, 0)),
            pl.BlockSpec((BT, D0), lambda i: (i, 0)),
            pl.BlockSpec((BT, D1), lambda i: (i, 0)),
            pl.BlockSpec((BT, D2), lambda i: (i, 0)),
            pl.BlockSpec((D0, D_PROJ), lambda i: (0, 0)),
            pl.BlockSpec((D1, D_PROJ), lambda i: (0, 0)),
            pl.BlockSpec((D2, D_PROJ), lambda i: (0, 0)),
        ],
        out_specs=pl.BlockSpec((BT, D_PROJ), lambda i: (i, 0)),
        out_shape=jax.ShapeDtypeStruct((N_TOKENS, D_PROJ), jnp.float32),
    )(idx2d, g0, g1, g2, p0t, p1t, p2t)


def kernel(input, emb_0, emb_1, emb_2, proj_0, proj_1, proj_2):
    idx_flat = input.reshape(-1).astype(jnp.int32)
    g0, g1, g2 = _sc_gather_kernel()(idx_flat, emb_0, emb_1, emb_2)
    out = _tc_project(idx_flat.reshape(-1, 1), g0, g1, g2,
                      proj_0.T, proj_1.T, proj_2.T)
    return out.reshape(input.shape + (D_PROJ,))


# speedup vs baseline: 1.0007x; 1.0007x over previous
import functools

import jax
import jax.numpy as jnp
from jax import lax
from jax.experimental import pallas as pl
from jax.experimental.pallas import tpu as pltpu
from jax.experimental.pallas import tpu_sc as plsc

D_PROJ = 128
CUT0 = 20000
CUT1 = 100000
CUT2 = 1000000
V0, D0 = 20000, 128
V1, D1 = 80000, 32
V2, D2 = 900000, 8

N_TOKENS = 4096 * 50
NC, NS = 2, 16
NW = NC * NS
TOK_PER_W = N_TOKENS // NW
CHUNK = 128
CHUNKS_PER_W = TOK_PER_W // CHUNK


def _sc_gather_body(idx_hbm, emb0, emb1, emb2, g0_hbm, g1_hbm, g2_hbm,
                    idx_v, i0_v, i1_v, i2_v, g0_v, g1_v, g2_v, sems):
    wid = lax.axis_index("s") * NC + lax.axis_index("c")

    def fire(ci, slot):
        base = (wid * CHUNKS_PER_W + ci) * CHUNK
        pltpu.sync_copy(idx_hbm.at[pl.ds(base, CHUNK)], idx_v.at[slot])

        def vec_body(j, c):
            v = idx_v[slot, pl.ds(j * 16, 16)]
            i0_v[slot, pl.ds(j * 16, 16)] = jnp.minimum(v, V0 - 1)
            i1_v[slot, pl.ds(j * 16, 16)] = jnp.clip(v - CUT0, 0, V1 - 1)
            i2_v[slot, pl.ds(j * 16, 16)] = jnp.clip(v - CUT1, 0, V2 - 1)
            return c

        lax.fori_loop(0, CHUNK // 16, vec_body, 0)
        pltpu.async_copy(emb0.at[i0_v.at[slot]], g0_v.at[slot], sems.at[slot, 0])
        pltpu.async_copy(emb1.at[i1_v.at[slot]], g1_v.at[slot], sems.at[slot, 1])
        pltpu.async_copy(emb2.at[i2_v.at[slot]], g2_v.at[slot], sems.at[slot, 2])

    def drain(ci, slot):
        base = (wid * CHUNKS_PER_W + ci) * CHUNK
        pltpu.make_async_copy(emb0.at[i0_v.at[slot]], g0_v.at[slot],
                              sems.at[slot, 0]).wait()
        pltpu.make_async_copy(emb1.at[i1_v.at[slot]], g1_v.at[slot],
                              sems.at[slot, 1]).wait()
        pltpu.make_async_copy(emb2.at[i2_v.at[slot]], g2_v.at[slot],
                              sems.at[slot, 2]).wait()
        pltpu.sync_copy(g0_v.at[slot], g0_hbm.at[pl.ds(base, CHUNK)])
        pltpu.sync_copy(g1_v.at[slot], g1_hbm.at[pl.ds(base, CHUNK)])
        pltpu.sync_copy(g2_v.at[slot], g2_hbm.at[pl.ds(base, CHUNK)])

    fire(0, 0)

    def pair_body(k, carry):
        fire(2 * k + 1, 1)
        drain(2 * k, 0)

        @pl.when(k < CHUNKS_PER_W // 2 - 1)
        def _():
            fire(2 * k + 2, 0)

        drain(2 * k + 1, 1)
        return carry

    lax.fori_loop(0, CHUNKS_PER_W // 2, pair_body, 0)


@functools.cache
def _sc_gather_kernel():
    return functools.partial(
        pl.kernel,
        mesh=plsc.VectorSubcoreMesh(core_axis_name="c", subcore_axis_name="s"),
        compiler_params=pltpu.CompilerParams(use_tc_tiling_on_sc=False),
        out_type=[
            jax.ShapeDtypeStruct((N_TOKENS, D0), jnp.float32),
            jax.ShapeDtypeStruct((N_TOKENS, D1), jnp.float32),
            jax.ShapeDtypeStruct((N_TOKENS, D2), jnp.float32),
        ],
        scratch_types=[
            pltpu.VMEM((2, CHUNK), jnp.int32),
            pltpu.VMEM((2, CHUNK), jnp.int32),
            pltpu.VMEM((2, CHUNK), jnp.int32),
            pltpu.VMEM((2, CHUNK), jnp.int32),
            pltpu.VMEM((2, CHUNK, D0), jnp.float32),
            pltpu.VMEM((2, CHUNK, D1), jnp.float32),
            pltpu.VMEM((2, CHUNK, D2), jnp.float32),
            pltpu.SemaphoreType.DMA((2, 3)),
        ],
    )(_sc_gather_body)


BT = 1024
N_BLOCKS = N_TOKENS // BT


def _tc_project_body(idx_ref, g0_ref, g1_ref, g2_ref, p0_ref, p1_ref, p2_ref,
                     out_ref):
    idx = idx_ref[...]
    m0 = (idx < CUT0).astype(jnp.float32)
    m1 = ((idx >= CUT0) & (idx < CUT1)).astype(jnp.float32)
    m2 = (idx >= CUT1).astype(jnp.float32)
    acc = jnp.dot(g0_ref[...] * m0, p0_ref[...],
                  preferred_element_type=jnp.float32)
    acc += jnp.dot(g1_ref[...] * m1, p1_ref[...],
                   preferred_element_type=jnp.float32)
    acc += jnp.dot(g2_ref[...] * m2, p2_ref[...],
                   preferred_element_type=jnp.float32)
    out_ref[...] = acc


def _tc_project(idx2d, g0, g1, g2, p0t, p1t, p2t):
    return pl.pallas_call(
        _tc_project_body,
        grid=(N_BLOCKS,),
        in_specs=[
            pl.BlockSpec((BT, 1), lambda i: (i, 0)),
            pl.BlockSpec((BT, D0), lambda i: (i, 0)),
            pl.BlockSpec((BT, D1), lambda i: (i, 0)),
            pl.BlockSpec((BT, D2), lambda i: (i, 0)),
            pl.BlockSpec((D0, D_PROJ), lambda i: (0, 0)),
            pl.BlockSpec((D1, D_PROJ), lambda i: (0, 0)),
            pl.BlockSpec((D2, D_PROJ), lambda i: (0, 0)),
        ],
        out_specs=pl.BlockSpec((BT, D_PROJ), lambda i: (i, 0)),
        out_shape=jax.ShapeDtypeStruct((N_TOKENS, D_PROJ), jnp.float32),
    )(idx2d, g0, g1, g2, p0t, p1t, p2t)


def kernel(input, emb_0, emb_1, emb_2, proj_0, proj_1, proj_2):
    idx_flat = input.reshape(-1).astype(jnp.int32)
    g0, g1, g2 = _sc_gather_kernel()(idx_flat, emb_0, emb_1, emb_2)
    out = _tc_project(idx_flat.reshape(-1, 1), g0, g1, g2,
                      proj_0.T, proj_1.T, proj_2.T)
    return out.reshape(input.shape + (D_PROJ,))


# speedup vs baseline: 8.0563x; 8.0509x over previous
import functools

import jax
import jax.numpy as jnp
from jax import lax
from jax.experimental import pallas as pl
from jax.experimental.pallas import tpu as pltpu
from jax.experimental.pallas import tpu_sc as plsc

D_PROJ = 128
CUT0 = 20000
CUT1 = 100000
CUT2 = 1000000
V0, D0 = 20000, 128
V1, D1 = 80000, 32
V2, D2 = 900000, 8

N_TOKENS = 4096 * 50
NC, NS = 2, 16
NW = NC * NS
TOK_PER_W = N_TOKENS // NW
CHUNK = 128
CHUNKS_PER_W = TOK_PER_W // CHUNK


def _sc_gather_body(idx_hbm, emb0, emb1, emb2, g0_hbm, g1_hbm, g2_hbm,
                    idx_v, i0_v, i1_v, i2_v, g0_v, g1_v, g2_v, sems):
    wid = lax.axis_index("s") * NC + lax.axis_index("c")

    def fire(ci, slot):
        base = (wid * CHUNKS_PER_W + ci) * CHUNK
        pltpu.sync_copy(idx_hbm.at[pl.ds(base, CHUNK)], idx_v.at[slot])

        def vec_body(j, c):
            v = idx_v[slot, pl.ds(j * 16, 16)]
            i0_v[slot, pl.ds(j * 16, 16)] = jnp.where(
                v < CUT0, v, v & (16384 - 1))
            i1_v[slot, pl.ds(j * 16, 16)] = jnp.where(
                (v >= CUT0) & (v < CUT1), v - CUT0, v & (65536 - 1))
            i2_v[slot, pl.ds(j * 16, 16)] = jnp.where(
                v >= CUT1, v - CUT1, v)
            return c

        lax.fori_loop(0, CHUNK // 16, vec_body, 0)
        pltpu.async_copy(emb0.at[i0_v.at[slot]], g0_v.at[slot], sems.at[slot, 0])
        pltpu.async_copy(emb1.at[i1_v.at[slot]], g1_v.at[slot], sems.at[slot, 1])
        pltpu.async_copy(emb2.at[i2_v.at[slot]], g2_v.at[slot], sems.at[slot, 2])

    def drain(ci, slot):
        base = (wid * CHUNKS_PER_W + ci) * CHUNK
        pltpu.make_async_copy(emb0.at[i0_v.at[slot]], g0_v.at[slot],
                              sems.at[slot, 0]).wait()
        pltpu.make_async_copy(emb1.at[i1_v.at[slot]], g1_v.at[slot],
                              sems.at[slot, 1]).wait()
        pltpu.make_async_copy(emb2.at[i2_v.at[slot]], g2_v.at[slot],
                              sems.at[slot, 2]).wait()
        pltpu.sync_copy(g0_v.at[slot], g0_hbm.at[pl.ds(base, CHUNK)])
        pltpu.sync_copy(g1_v.at[slot], g1_hbm.at[pl.ds(base, CHUNK)])
        pltpu.sync_copy(g2_v.at[slot], g2_hbm.at[pl.ds(base, CHUNK)])

    fire(0, 0)

    def pair_body(k, carry):
        fire(2 * k + 1, 1)
        drain(2 * k, 0)

        @pl.when(k < CHUNKS_PER_W // 2 - 1)
        def _():
            fire(2 * k + 2, 0)

        drain(2 * k + 1, 1)
        return carry

    lax.fori_loop(0, CHUNKS_PER_W // 2, pair_body, 0)


@functools.cache
def _sc_gather_kernel():
    return functools.partial(
        pl.kernel,
        mesh=plsc.VectorSubcoreMesh(core_axis_name="c", subcore_axis_name="s"),
        compiler_params=pltpu.CompilerParams(use_tc_tiling_on_sc=False),
        out_type=[
            jax.ShapeDtypeStruct((N_TOKENS, D0), jnp.float32),
            jax.ShapeDtypeStruct((N_TOKENS, D1), jnp.float32),
            jax.ShapeDtypeStruct((N_TOKENS, D2), jnp.float32),
        ],
        scratch_types=[
            pltpu.VMEM((2, CHUNK), jnp.int32),
            pltpu.VMEM((2, CHUNK), jnp.int32),
            pltpu.VMEM((2, CHUNK), jnp.int32),
            pltpu.VMEM((2, CHUNK), jnp.int32),
            pltpu.VMEM((2, CHUNK, D0), jnp.float32),
            pltpu.VMEM((2, CHUNK, D1), jnp.float32),
            pltpu.VMEM((2, CHUNK, D2), jnp.float32),
            pltpu.SemaphoreType.DMA((2, 3)),
        ],
    )(_sc_gather_body)


BT = 1024
N_BLOCKS = N_TOKENS // BT


def _tc_project_body(idx_ref, g0_ref, g1_ref, g2_ref, p0_ref, p1_ref, p2_ref,
                     out_ref):
    idx = idx_ref[...]
    m0 = (idx < CUT0).astype(jnp.float32)
    m1 = ((idx >= CUT0) & (idx < CUT1)).astype(jnp.float32)
    m2 = (idx >= CUT1).astype(jnp.float32)
    acc = jnp.dot(g0_ref[...] * m0, p0_ref[...],
                  preferred_element_type=jnp.float32)
    acc += jnp.dot(g1_ref[...] * m1, p1_ref[...],
                   preferred_element_type=jnp.float32)
    acc += jnp.dot(g2_ref[...] * m2, p2_ref[...],
                   preferred_element_type=jnp.float32)
    out_ref[...] = acc


def _tc_project(idx2d, g0, g1, g2, p0t, p1t, p2t):
    return pl.pallas_call(
        _tc_project_body,
        grid=(N_BLOCKS,),
        in_specs=[
            pl.BlockSpec((BT, 1), lambda i: (i, 0)),
            pl.BlockSpec((BT, D0), lambda i: (i, 0)),
            pl.BlockSpec((BT, D1), lambda i: (i, 0)),
            pl.BlockSpec((BT, D2), lambda i: (i, 0)),
            pl.BlockSpec((D0, D_PROJ), lambda i: (0, 0)),
            pl.BlockSpec((D1, D_PROJ), lambda i: (0, 0)),
            pl.BlockSpec((D2, D_PROJ), lambda i: (0, 0)),
        ],
        out_specs=pl.BlockSpec((BT, D_PROJ), lambda i: (i, 0)),
        out_shape=jax.ShapeDtypeStruct((N_TOKENS, D_PROJ), jnp.float32),
    )(idx2d, g0, g1, g2, p0t, p1t, p2t)


def kernel(input, emb_0, emb_1, emb_2, proj_0, proj_1, proj_2):
    idx_flat = input.reshape(-1).astype(jnp.int32)
    g0, g1, g2 = _sc_gather_kernel()(idx_flat, emb_0, emb_1, emb_2)
    out = _tc_project(idx_flat.reshape(-1, 1), g0, g1, g2,
                      proj_0.T, proj_1.T, proj_2.T)
    return out.reshape(input.shape + (D_PROJ,))


# speedup vs baseline: 8.8477x; 1.0982x over previous
import functools

import jax
import jax.numpy as jnp
from jax import lax
from jax.experimental import pallas as pl
from jax.experimental.pallas import tpu as pltpu
from jax.experimental.pallas import tpu_sc as plsc

D_PROJ = 128
CUT0 = 20000
CUT1 = 100000
CUT2 = 1000000
V0, D0 = 20000, 128
V1, D1 = 80000, 32
V2, D2 = 900000, 8

N_TOKENS = 4096 * 50
NC, NS = 2, 16
NW = NC * NS
TOK_PER_W = N_TOKENS // NW
CHUNK = 128
CHUNKS_PER_W = TOK_PER_W // CHUNK


def _sc_gather_body(idx_hbm, emb0, emb1, emb2, g0_hbm, g1_hbm, g2_hbm,
                    idx_v, i0_v, i1_v, i2_v, g0_v, g1_v, g2_v, sems):
    wid = lax.axis_index("s") * NC + lax.axis_index("c")

    def fire(ci, slot):
        base = (wid * CHUNKS_PER_W + ci) * CHUNK
        pltpu.sync_copy(idx_hbm.at[pl.ds(base, CHUNK)], idx_v.at[slot])

        def vec_body(j, c):
            v = idx_v[slot, pl.ds(j * 16, 16)]
            i0_v[slot, pl.ds(j * 16, 16)] = jnp.where(
                v < CUT0, v, v & (16384 - 1))
            i1_v[slot, pl.ds(j * 16, 16)] = jnp.where(
                (v >= CUT0) & (v < CUT1), v - CUT0, v & (65536 - 1))
            i2_v[slot, pl.ds(j * 16, 16)] = jnp.where(
                v >= CUT1, v - CUT1, v)
            return c

        lax.fori_loop(0, CHUNK // 16, vec_body, 0)
        pltpu.async_copy(emb0.at[i0_v.at[slot]], g0_v.at[slot], sems.at[slot, 0])
        pltpu.async_copy(emb1.at[i1_v.at[slot]], g1_v.at[slot], sems.at[slot, 1])
        pltpu.async_copy(emb2.at[i2_v.at[slot]], g2_v.at[slot], sems.at[slot, 2])

    def drain(ci, slot):
        g = wid * CHUNKS_PER_W + ci
        base = g * CHUNK
        b = g // 8
        c = g % 8
        pltpu.make_async_copy(emb0.at[i0_v.at[slot]], g0_v.at[slot],
                              sems.at[slot, 0]).wait()
        pltpu.make_async_copy(emb1.at[i1_v.at[slot]], g1_v.at[slot],
                              sems.at[slot, 1]).wait()
        pltpu.make_async_copy(emb2.at[i2_v.at[slot]], g2_v.at[slot],
                              sems.at[slot, 2]).wait()
        pltpu.sync_copy(g0_v.at[slot], g0_hbm.at[pl.ds(base, CHUNK)])
        pltpu.sync_copy(
            g1_v.at[slot],
            g1_hbm.at[pl.ds(b * 256 + (c % 2) * CHUNK, CHUNK),
                      pl.ds(32 * (c // 2), 32)])
        pltpu.sync_copy(
            g2_v.at[slot, pl.ds(0, 64)],
            g2_hbm.at[pl.ds(b * 64, 64), pl.ds(8 * (2 * c), 8)])
        pltpu.sync_copy(
            g2_v.at[slot, pl.ds(64, 64)],
            g2_hbm.at[pl.ds(b * 64, 64), pl.ds(8 * (2 * c + 1), 8)])

    fire(0, 0)

    def pair_body(k, carry):
        fire(2 * k + 1, 1)
        drain(2 * k, 0)

        @pl.when(k < CHUNKS_PER_W // 2 - 1)
        def _():
            fire(2 * k + 2, 0)

        drain(2 * k + 1, 1)
        return carry

    lax.fori_loop(0, CHUNKS_PER_W // 2, pair_body, 0)


@functools.cache
def _sc_gather_kernel():
    return functools.partial(
        pl.kernel,
        mesh=plsc.VectorSubcoreMesh(core_axis_name="c", subcore_axis_name="s"),
        compiler_params=pltpu.CompilerParams(use_tc_tiling_on_sc=False),
        out_type=[
            jax.ShapeDtypeStruct((N_TOKENS, D0), jnp.float32),
            jax.ShapeDtypeStruct((N_TOKENS // 4, 128), jnp.float32),
            jax.ShapeDtypeStruct((N_TOKENS // 16, 128), jnp.float32),
        ],
        scratch_types=[
            pltpu.VMEM((2, CHUNK), jnp.int32),
            pltpu.VMEM((2, CHUNK), jnp.int32),
            pltpu.VMEM((2, CHUNK), jnp.int32),
            pltpu.VMEM((2, CHUNK), jnp.int32),
            pltpu.VMEM((2, CHUNK, D0), jnp.float32),
            pltpu.VMEM((2, CHUNK, D1), jnp.float32),
            pltpu.VMEM((2, CHUNK, D2), jnp.float32),
            pltpu.SemaphoreType.DMA((2, 3)),
        ],
    )(_sc_gather_body)


BT = 1024
N_BLOCKS = N_TOKENS // BT


def _tc_project_body(idx_ref, g0_ref, g1_ref, g2_ref, p0_ref, p1_ref, p2_ref,
                     out_ref):
    idx = idx_ref[...]
    m0 = (idx < CUT0).astype(jnp.float32)
    m1 = ((idx >= CUT0) & (idx < CUT1)).astype(jnp.float32)
    m2 = (idx >= CUT1).astype(jnp.float32)
    acc = jnp.dot(g0_ref[...] * m0, p0_ref[...],
                  preferred_element_type=jnp.float32)
    g1p = g1_ref[...]
    acc += jnp.concatenate(
        [jnp.dot(g1p[:, 32 * q:32 * q + 32] * m1[256 * q:256 * q + 256],
                 p1_ref[...], preferred_element_type=jnp.float32)
         for q in range(4)], axis=0)
    g2p = g2_ref[...]
    acc += jnp.concatenate(
        [jnp.dot(g2p[:, 8 * q:8 * q + 8] * m2[64 * q:64 * q + 64],
                 p2_ref[...], preferred_element_type=jnp.float32)
         for q in range(16)], axis=0)
    out_ref[...] = acc


def _tc_project(idx2d, g0, g1, g2, p0t, p1t, p2t):
    return pl.pallas_call(
        _tc_project_body,
        grid=(N_BLOCKS,),
        in_specs=[
            pl.BlockSpec((BT, 1), lambda i: (i, 0)),
            pl.BlockSpec((BT, D0), lambda i: (i, 0)),
            pl.BlockSpec((BT // 4, 128), lambda i: (i, 0)),
            pl.BlockSpec((BT // 16, 128), lambda i: (i, 0)),
            pl.BlockSpec((D0, D_PROJ), lambda i: (0, 0)),
            pl.BlockSpec((D1, D_PROJ), lambda i: (0, 0)),
            pl.BlockSpec((D2, D_PROJ), lambda i: (0, 0)),
        ],
        out_specs=pl.BlockSpec((BT, D_PROJ), lambda i: (i, 0)),
        out_shape=jax.ShapeDtypeStruct((N_TOKENS, D_PROJ), jnp.float32),
    )(idx2d, g0, g1, g2, p0t, p1t, p2t)


def kernel(input, emb_0, emb_1, emb_2, proj_0, proj_1, proj_2):
    idx_flat = input.reshape(-1).astype(jnp.int32)
    g0, g1, g2 = _sc_gather_kernel()(idx_flat, emb_0, emb_1, emb_2)
    out = _tc_project(idx_flat.reshape(-1, 1), g0, g1, g2,
                      proj_0.T, proj_1.T, proj_2.T)
    return out.reshape(input.shape + (D_PROJ,))


# speedup vs baseline: 11.7809x; 1.3315x over previous
import functools

import jax
import jax.numpy as jnp
from jax import lax
from jax.experimental import pallas as pl
from jax.experimental.pallas import tpu as pltpu
from jax.experimental.pallas import tpu_sc as plsc

D_PROJ = 128
CUT0 = 20000
CUT1 = 100000
CUT2 = 1000000
V0, D0 = 20000, 128
V1, D1 = 80000, 32
V2, D2 = 900000, 8

N_TOKENS = 4096 * 50
ZPAD = 2048
NC, NS = 2, 16
NW = NC * NS
TOK_PER_W = N_TOKENS // NW
CHUNK = 128
CHUNKS_PER_W = TOK_PER_W // CHUNK


def _sc_gather_body(idx_hbm, emb0, emb1, emb2, g0_hbm, g1_hbm, g2_hbm,
                    idx_v, i0_v, i1_v, i2_v, g0_v, g1_v, g2_v, sems):
    wid = lax.axis_index("s") * NC + lax.axis_index("c")

    def fire(ci, slot):
        base = (wid * CHUNKS_PER_W + ci) * CHUNK
        pltpu.sync_copy(idx_hbm.at[pl.ds(base, CHUNK)], idx_v.at[slot])

        def vec_body(j, c):
            v = idx_v[slot, pl.ds(j * 16, 16)]
            z = v & (ZPAD - 1)
            i0_v[slot, pl.ds(j * 16, 16)] = jnp.where(v < CUT0, v, V0 + z)
            i1_v[slot, pl.ds(j * 16, 16)] = jnp.where(
                (v >= CUT0) & (v < CUT1), v - CUT0, V1 + z)
            i2_v[slot, pl.ds(j * 16, 16)] = jnp.where(
                v >= CUT1, v - CUT1, V2 + z)
            return c

        lax.fori_loop(0, CHUNK // 16, vec_body, 0)
        pltpu.async_copy(emb0.at[i0_v.at[slot]], g0_v.at[slot], sems.at[slot, 0])
        pltpu.async_copy(emb1.at[i1_v.at[slot]], g1_v.at[slot], sems.at[slot, 1])
        pltpu.async_copy(emb2.at[i2_v.at[slot]], g2_v.at[slot], sems.at[slot, 2])

    def drain(ci, slot):
        g = wid * CHUNKS_PER_W + ci
        base = g * CHUNK
        b = g // 8
        c = g % 8
        pltpu.make_async_copy(emb0.at[i0_v.at[slot]], g0_v.at[slot],
                              sems.at[slot, 0]).wait()
        pltpu.make_async_copy(emb1.at[i1_v.at[slot]], g1_v.at[slot],
                              sems.at[slot, 1]).wait()
        pltpu.make_async_copy(emb2.at[i2_v.at[slot]], g2_v.at[slot],
                              sems.at[slot, 2]).wait()
        pltpu.sync_copy(g0_v.at[slot], g0_hbm.at[pl.ds(base, CHUNK)])
        pltpu.sync_copy(
            g1_v.at[slot],
            g1_hbm.at[pl.ds(b * 256 + (c % 2) * CHUNK, CHUNK),
                      pl.ds(32 * (c // 2), 32)])
        pltpu.sync_copy(
            g2_v.at[slot, pl.ds(0, 64)],
            g2_hbm.at[pl.ds(b * 64, 64), pl.ds(8 * (2 * c), 8)])
        pltpu.sync_copy(
            g2_v.at[slot, pl.ds(64, 64)],
            g2_hbm.at[pl.ds(b * 64, 64), pl.ds(8 * (2 * c + 1), 8)])

    fire(0, 0)

    def pair_body(k, carry):
        fire(2 * k + 1, 1)
        drain(2 * k, 0)

        @pl.when(k < CHUNKS_PER_W // 2 - 1)
        def _():
            fire(2 * k + 2, 0)

        drain(2 * k + 1, 1)
        return carry

    lax.fori_loop(0, CHUNKS_PER_W // 2, pair_body, 0)


@functools.cache
def _sc_gather_kernel():
    return functools.partial(
        pl.kernel,
        mesh=plsc.VectorSubcoreMesh(core_axis_name="c", subcore_axis_name="s"),
        compiler_params=pltpu.CompilerParams(use_tc_tiling_on_sc=False),
        out_type=[
            jax.ShapeDtypeStruct((N_TOKENS, D0), jnp.float32),
            jax.ShapeDtypeStruct((N_TOKENS // 4, 128), jnp.float32),
            jax.ShapeDtypeStruct((N_TOKENS // 16, 128), jnp.float32),
        ],
        scratch_types=[
            pltpu.VMEM((2, CHUNK), jnp.int32),
            pltpu.VMEM((2, CHUNK), jnp.int32),
            pltpu.VMEM((2, CHUNK), jnp.int32),
            pltpu.VMEM((2, CHUNK), jnp.int32),
            pltpu.VMEM((2, CHUNK, D0), jnp.float32),
            pltpu.VMEM((2, CHUNK, D1), jnp.float32),
            pltpu.VMEM((2, CHUNK, D2), jnp.float32),
            pltpu.SemaphoreType.DMA((2, 3)),
        ],
    )(_sc_gather_body)


BT = 1024
N_BLOCKS = N_TOKENS // BT


def _tc_project_body(g0_ref, g1_ref, g2_ref, p0_ref, p1_ref, p2_ref, out_ref):
    acc = jnp.dot(g0_ref[...], p0_ref[...],
                  preferred_element_type=jnp.float32)
    g1p = g1_ref[...]
    acc += jnp.concatenate(
        [jnp.dot(g1p[:, 32 * q:32 * q + 32], p1_ref[...],
                 preferred_element_type=jnp.float32)
         for q in range(4)], axis=0)
    g2p = g2_ref[...]
    acc += jnp.concatenate(
        [jnp.dot(g2p[:, 8 * q:8 * q + 8], p2_ref[...],
                 preferred_element_type=jnp.float32)
         for q in range(16)], axis=0)
    out_ref[...] = acc


def _tc_project(g0, g1, g2, p0t, p1t, p2t):
    return pl.pallas_call(
        _tc_project_body,
        grid=(N_BLOCKS,),
        in_specs=[
            pl.BlockSpec((BT, D0), lambda i: (i, 0)),
            pl.BlockSpec((BT // 4, 128), lambda i: (i, 0)),
            pl.BlockSpec((BT // 16, 128), lambda i: (i, 0)),
            pl.BlockSpec((D0, D_PROJ), lambda i: (0, 0)),
            pl.BlockSpec((D1, D_PROJ), lambda i: (0, 0)),
            pl.BlockSpec((D2, D_PROJ), lambda i: (0, 0)),
        ],
        out_specs=pl.BlockSpec((BT, D_PROJ), lambda i: (i, 0)),
        out_shape=jax.ShapeDtypeStruct((N_TOKENS, D_PROJ), jnp.float32),
    )(g0, g1, g2, p0t, p1t, p2t)


def kernel(input, emb_0, emb_1, emb_2, proj_0, proj_1, proj_2):
    idx_flat = input.T.reshape(-1).astype(jnp.int32)
    e0 = jnp.concatenate(
        [emb_0.reshape(-1), jnp.zeros(ZPAD * D0, jnp.float32)]
    ).reshape(V0 + ZPAD, D0)
    e1 = jnp.concatenate(
        [emb_1.reshape(-1), jnp.zeros(ZPAD * D1, jnp.float32)]
    ).reshape(V1 + ZPAD, D1)
    e2 = jnp.concatenate(
        [emb_2.reshape(-1), jnp.zeros(ZPAD * D2, jnp.float32)]
    ).reshape(V2 + ZPAD, D2)
    g0, g1, g2 = _sc_gather_kernel()(idx_flat, e0, e1, e2)
    out = _tc_project(g0, g1, g2, proj_0.T, proj_1.T, proj_2.T)
    return out.reshape(50, 4096, D_PROJ).transpose(1, 0, 2)


# speedup vs baseline: 12.1559x; 1.0318x over previous
import functools

import jax
import jax.numpy as jnp
from jax import lax
from jax.experimental import pallas as pl
from jax.experimental.pallas import tpu as pltpu
from jax.experimental.pallas import tpu_sc as plsc

D_PROJ = 128
CUT0 = 20000
CUT1 = 100000
V0, D0 = 20000, 128
V1, D1 = 80000, 32
V2, D2 = 900000, 8

N_TOKENS = 4096 * 50
ZPAD = 2048
NC, NS = 2, 16
NW = NC * NS
TOK_PER_W = N_TOKENS // NW
CHUNK = 128
CHUNKS_PER_W = TOK_PER_W // CHUNK


def _sc_gather01_body(idx_hbm, emb0, emb1, g0_hbm, g1_hbm,
                      idx_v, i0_v, i1_v, g0_v, g1_v, sems):
    wid = lax.axis_index("s") * NC + lax.axis_index("c")

    def fire(ci, slot):
        base = (wid * CHUNKS_PER_W + ci) * CHUNK
        pltpu.sync_copy(idx_hbm.at[pl.ds(base, CHUNK)], idx_v.at[slot])

        def vec_body(j, c):
            v = idx_v[slot, pl.ds(j * 16, 16)]
            z = v & (ZPAD - 1)
            i0_v[slot, pl.ds(j * 16, 16)] = jnp.where(v < CUT0, v, V0 + z)
            i1_v[slot, pl.ds(j * 16, 16)] = jnp.where(
                (v >= CUT0) & (v < CUT1), v - CUT0, V1 + z)
            return c

        lax.fori_loop(0, CHUNK // 16, vec_body, 0)
        pltpu.async_copy(emb0.at[i0_v.at[slot]], g0_v.at[slot],
                         sems.at[slot, 0])
        pltpu.async_copy(emb1.at[i1_v.at[slot]], g1_v.at[slot],
                         sems.at[slot, 1])

    def drain(ci, slot):
        g = wid * CHUNKS_PER_W + ci
        base = g * CHUNK
        b = g // 8
        c = g % 8
        pltpu.make_async_copy(emb0.at[i0_v.at[slot]], g0_v.at[slot],
                              sems.at[slot, 0]).wait()
        pltpu.make_async_copy(emb1.at[i1_v.at[slot]], g1_v.at[slot],
                              sems.at[slot, 1]).wait()
        pltpu.sync_copy(g0_v.at[slot], g0_hbm.at[pl.ds(base, CHUNK)])
        pltpu.sync_copy(
            g1_v.at[slot],
            g1_hbm.at[pl.ds(b * 256 + (c % 2) * CHUNK, CHUNK),
                      pl.ds(32 * (c // 2), 32)])

    fire(0, 0)

    def pair_body(k, carry):
        fire(2 * k + 1, 1)
        drain(2 * k, 0)

        @pl.when(k < CHUNKS_PER_W // 2 - 1)
        def _():
            fire(2 * k + 2, 0)

        drain(2 * k + 1, 1)
        return carry

    lax.fori_loop(0, CHUNKS_PER_W // 2, pair_body, 0)


def _sc_gather2_body(idx_hbm, emb2, g2_hbm, idx_v, i2_v, g2_v, sems):
    wid = lax.axis_index("s") * NC + lax.axis_index("c")

    def fire(ci, slot):
        base = (wid * CHUNKS_PER_W + ci) * CHUNK
        pltpu.sync_copy(idx_hbm.at[pl.ds(base, CHUNK)], idx_v.at[slot])

        def vec_body(j, c):
            v = idx_v[slot, pl.ds(j * 16, 16)]
            z = v & (ZPAD - 1)
            i2_v[slot, pl.ds(j * 16, 16)] = jnp.where(
                v >= CUT1, v - CUT1, V2 + z)
            return c

        lax.fori_loop(0, CHUNK // 16, vec_body, 0)
        pltpu.async_copy(emb2.at[i2_v.at[slot]], g2_v.at[slot],
                         sems.at[slot])

    def drain(ci, slot):
        g = wid * CHUNKS_PER_W + ci
        b = g // 8
        c = g % 8
        pltpu.make_async_copy(emb2.at[i2_v.at[slot]], g2_v.at[slot],
                              sems.at[slot]).wait()
        pltpu.sync_copy(
            g2_v.at[slot, pl.ds(0, 64)],
            g2_hbm.at[pl.ds(b * 64, 64), pl.ds(8 * (2 * c), 8)])
        pltpu.sync_copy(
            g2_v.at[slot, pl.ds(64, 64)],
            g2_hbm.at[pl.ds(b * 64, 64), pl.ds(8 * (2 * c + 1), 8)])

    fire(0, 0)

    def pair_body(k, carry):
        fire(2 * k + 1, 1)
        drain(2 * k, 0)

        @pl.when(k < CHUNKS_PER_W // 2 - 1)
        def _():
            fire(2 * k + 2, 0)

        drain(2 * k + 1, 1)
        return carry

    lax.fori_loop(0, CHUNKS_PER_W // 2, pair_body, 0)


@functools.cache
def _sc_gather01_kernel():
    return functools.partial(
        pl.kernel,
        mesh=plsc.VectorSubcoreMesh(core_axis_name="c", subcore_axis_name="s"),
        compiler_params=pltpu.CompilerParams(use_tc_tiling_on_sc=False),
        out_type=[
            jax.ShapeDtypeStruct((N_TOKENS, D0), jnp.float32),
            jax.ShapeDtypeStruct((N_TOKENS // 4, 128), jnp.float32),
        ],
        scratch_types=[
            pltpu.VMEM((2, CHUNK), jnp.int32),
            pltpu.VMEM((2, CHUNK), jnp.int32),
            pltpu.VMEM((2, CHUNK), jnp.int32),
            pltpu.VMEM((2, CHUNK, D0), jnp.float32),
            pltpu.VMEM((2, CHUNK, D1), jnp.float32),
            pltpu.SemaphoreType.DMA((2, 2)),
        ],
    )(_sc_gather01_body)


@functools.cache
def _sc_gather2_kernel():
    return functools.partial(
        pl.kernel,
        mesh=plsc.VectorSubcoreMesh(core_axis_name="c", subcore_axis_name="s"),
        compiler_params=pltpu.CompilerParams(use_tc_tiling_on_sc=False),
        out_type=[
            jax.ShapeDtypeStruct((N_TOKENS // 16, 128), jnp.float32),
        ],
        scratch_types=[
            pltpu.VMEM((2, CHUNK), jnp.int32),
            pltpu.VMEM((2, CHUNK), jnp.int32),
            pltpu.VMEM((2, CHUNK, D2), jnp.float32),
            pltpu.SemaphoreType.DMA((2,)),
        ],
    )(_sc_gather2_body)


BT = 1024
N_BLOCKS = N_TOKENS // BT


def _tc_project_body(g0_ref, g1_ref, g2_ref, p0_ref, p1_ref, p2_ref, out_ref):
    acc = jnp.dot(g0_ref[...], p0_ref[...],
                  preferred_element_type=jnp.float32)
    g1p = g1_ref[...]
    acc += jnp.concatenate(
        [jnp.dot(g1p[:, 32 * q:32 * q + 32], p1_ref[...],
                 preferred_element_type=jnp.float32)
         for q in range(4)], axis=0)
    g2p = g2_ref[...]
    acc += jnp.concatenate(
        [jnp.dot(g2p[:, 8 * q:8 * q + 8], p2_ref[...],
                 preferred_element_type=jnp.float32)
         for q in range(16)], axis=0)
    out_ref[...] = acc


def _tc_project(g0, g1, g2, p0t, p1t, p2t):
    return pl.pallas_call(
        _tc_project_body,
        grid=(N_BLOCKS,),
        in_specs=[
            pl.BlockSpec((BT, D0), lambda i: (i, 0)),
            pl.BlockSpec((BT // 4, 128), lambda i: (i, 0)),
            pl.BlockSpec((BT // 16, 128), lambda i: (i, 0)),
            pl.BlockSpec((D0, D_PROJ), lambda i: (0, 0)),
            pl.BlockSpec((D1, D_PROJ), lambda i: (0, 0)),
            pl.BlockSpec((D2, D_PROJ), lambda i: (0, 0)),
        ],
        out_specs=pl.BlockSpec((BT, D_PROJ), lambda i: (i, 0)),
        out_shape=jax.ShapeDtypeStruct((N_TOKENS, D_PROJ), jnp.float32),
    )(g0, g1, g2, p0t, p1t, p2t)


def kernel(input, emb_0, emb_1, emb_2, proj_0, proj_1, proj_2):
    idx_flat = input.T.reshape(-1).astype(jnp.int32)
    e0 = jnp.concatenate(
        [emb_0, jnp.zeros((ZPAD, D0), jnp.float32)], axis=0)
    e1 = jnp.concatenate(
        [emb_1.reshape(-1), jnp.zeros(ZPAD * D1, jnp.float32)]
    ).reshape(V1 + ZPAD, D1)
    e2 = jnp.concatenate(
        [emb_2.reshape(-1), jnp.zeros(ZPAD * D2, jnp.float32)]
    ).reshape(V2 + ZPAD, D2)
    g0, g1 = _sc_gather01_kernel()(idx_flat, e0, e1)
    (g2,) = _sc_gather2_kernel()(idx_flat, e2)
    out = _tc_project(g0, g1, g2, proj_0.T, proj_1.T, proj_2.T)
    return out.reshape(50, 4096, D_PROJ).transpose(1, 0, 2)


# speedup vs baseline: 13.0929x; 1.0771x over previous
import functools

import jax
import jax.numpy as jnp
from jax import lax
from jax.experimental import pallas as pl
from jax.experimental.pallas import tpu as pltpu
from jax.experimental.pallas import tpu_sc as plsc

D_PROJ = 128
CUT0 = 20000
CUT1 = 100000
V0, D0 = 20000, 128
V1, D1 = 80000, 32
V2, D2 = 900000, 8

N_TOKENS = 4096 * 50
ZPAD = 2048
NC, NS = 2, 16
NW = NC * NS
TOK_PER_W = N_TOKENS // NW
CHUNK = 128
CHUNKS_PER_W = TOK_PER_W // CHUNK


def _sc_gather01_body(idx_hbm, emb0, emb1, g0_hbm, g1_hbm,
                      idx_v, i0_v, i1_v, g0_v, g1_v, sems):
    wid = lax.axis_index("s") * NC + lax.axis_index("c")

    def fire(ci, slot):
        base = (wid * CHUNKS_PER_W + ci) * CHUNK
        pltpu.sync_copy(idx_hbm.at[pl.ds(base, CHUNK)], idx_v.at[slot])

        def vec_body(j, c):
            v = idx_v[slot, pl.ds(j * 16, 16)]
            z = v & (ZPAD - 1)
            i0_v[slot, pl.ds(j * 16, 16)] = jnp.where(v < CUT0, v, V0 + z)
            i1_v[slot, pl.ds(j * 16, 16)] = jnp.where(
                (v >= CUT0) & (v < CUT1), v - CUT0, V1 + z)
            return c

        lax.fori_loop(0, CHUNK // 16, vec_body, 0)
        pltpu.async_copy(emb0.at[i0_v.at[slot]], g0_v.at[slot],
                         sems.at[slot, 0])
        pltpu.async_copy(emb1.at[i1_v.at[slot]], g1_v.at[slot],
                         sems.at[slot, 1])

    def drain(ci, slot):
        g = wid * CHUNKS_PER_W + ci
        base = g * CHUNK
        b = g // 8
        c = g % 8
        pltpu.make_async_copy(emb0.at[i0_v.at[slot]], g0_v.at[slot],
                              sems.at[slot, 0]).wait()
        pltpu.make_async_copy(emb1.at[i1_v.at[slot]], g1_v.at[slot],
                              sems.at[slot, 1]).wait()
        pltpu.sync_copy(g0_v.at[slot], g0_hbm.at[pl.ds(base, CHUNK)])
        pltpu.sync_copy(
            g1_v.at[slot],
            g1_hbm.at[pl.ds(b * 256 + (c % 2) * CHUNK, CHUNK),
                      pl.ds(32 * (c // 2), 32)])

    fire(0, 0)

    def pair_body(k, carry):
        fire(2 * k + 1, 1)
        drain(2 * k, 0)

        @pl.when(k < CHUNKS_PER_W // 2 - 1)
        def _():
            fire(2 * k + 2, 0)

        drain(2 * k + 1, 1)
        return carry

    lax.fori_loop(0, CHUNKS_PER_W // 2, pair_body, 0)


def _sc_gather2_body(idx_hbm, emb2, g2_hbm, idx_v, i2_v, g2_v, sems):
    wid = lax.axis_index("s") * NC + lax.axis_index("c")

    def fire(ci, slot):
        base = (wid * CHUNKS_PER_W + ci) * CHUNK
        pltpu.sync_copy(idx_hbm.at[pl.ds(base, CHUNK)], idx_v.at[slot])

        def vec_body(j, c):
            v = idx_v[slot, pl.ds(j * 16, 16)]
            z = v & (ZPAD - 1)
            i2_v[slot, pl.ds(j * 16, 16)] = jnp.where(
                v >= CUT1, v - CUT1, V2 + z)
            return c

        lax.fori_loop(0, CHUNK // 16, vec_body, 0)
        pltpu.async_copy(emb2.at[i2_v.at[slot]], g2_v.at[slot],
                         sems.at[slot])

    def drain(ci, slot):
        g = wid * CHUNKS_PER_W + ci
        b = g // 8
        c = g % 8
        pltpu.make_async_copy(emb2.at[i2_v.at[slot]], g2_v.at[slot],
                              sems.at[slot]).wait()
        pltpu.sync_copy(
            g2_v.at[slot, pl.ds(0, 64)],
            g2_hbm.at[pl.ds(b * 64, 64), pl.ds(8 * (2 * c), 8)])
        pltpu.sync_copy(
            g2_v.at[slot, pl.ds(64, 64)],
            g2_hbm.at[pl.ds(b * 64, 64), pl.ds(8 * (2 * c + 1), 8)])

    fire(0, 0)

    def pair_body(k, carry):
        fire(2 * k + 1, 1)
        drain(2 * k, 0)

        @pl.when(k < CHUNKS_PER_W // 2 - 1)
        def _():
            fire(2 * k + 2, 0)

        drain(2 * k + 1, 1)
        return carry

    lax.fori_loop(0, CHUNKS_PER_W // 2, pair_body, 0)


@functools.cache
def _sc_gather01_kernel():
    return functools.partial(
        pl.kernel,
        mesh=plsc.VectorSubcoreMesh(core_axis_name="c", subcore_axis_name="s"),
        compiler_params=pltpu.CompilerParams(use_tc_tiling_on_sc=False),
        out_type=[
            jax.ShapeDtypeStruct((N_TOKENS, D0), jnp.float32),
            jax.ShapeDtypeStruct((N_TOKENS // 4, 128), jnp.float32),
        ],
        scratch_types=[
            pltpu.VMEM((2, CHUNK), jnp.int32),
            pltpu.VMEM((2, CHUNK), jnp.int32),
            pltpu.VMEM((2, CHUNK), jnp.int32),
            pltpu.VMEM((2, CHUNK, D0), jnp.float32),
            pltpu.VMEM((2, CHUNK, D1), jnp.float32),
            pltpu.SemaphoreType.DMA((2, 2)),
        ],
    )(_sc_gather01_body)


@functools.cache
def _sc_gather2_kernel():
    return functools.partial(
        pl.kernel,
        mesh=plsc.VectorSubcoreMesh(core_axis_name="c", subcore_axis_name="s"),
        compiler_params=pltpu.CompilerParams(use_tc_tiling_on_sc=False),
        out_type=[
            jax.ShapeDtypeStruct((N_TOKENS // 16, 128), jnp.float32),
        ],
        scratch_types=[
            pltpu.VMEM((2, CHUNK), jnp.int32),
            pltpu.VMEM((2, CHUNK), jnp.int32),
            pltpu.VMEM((2, CHUNK, D2), jnp.float32),
            pltpu.SemaphoreType.DMA((2,)),
        ],
    )(_sc_gather2_body)


BT = 2048
N_BLOCKS = N_TOKENS // BT


def _tc_project_body(g0_ref, g1_ref, g2_ref, p0_ref, p1_ref, p2_ref, out_ref):
    acc = jnp.dot(g0_ref[...], p0_ref[...],
                  preferred_element_type=jnp.float32)
    g1p = g1_ref[...]
    acc += jnp.concatenate(
        [jnp.dot(g1p[256 * u:256 * u + 256, 32 * q:32 * q + 32], p1_ref[...],
                 preferred_element_type=jnp.float32)
         for u in range(BT // 1024) for q in range(4)], axis=0)
    g2p = g2_ref[...]
    acc += jnp.concatenate(
        [jnp.dot(g2p[64 * u:64 * u + 64, 8 * q:8 * q + 8], p2_ref[...],
                 preferred_element_type=jnp.float32)
         for u in range(BT // 1024) for q in range(16)], axis=0)
    out_ref[...] = acc


def _tc_project(g0, g1, g2, p0t, p1t, p2t):
    return pl.pallas_call(
        _tc_project_body,
        grid=(N_BLOCKS,),
        in_specs=[
            pl.BlockSpec((BT, D0), lambda i: (i, 0)),
            pl.BlockSpec((BT // 4, 128), lambda i: (i, 0)),
            pl.BlockSpec((BT // 16, 128), lambda i: (i, 0)),
            pl.BlockSpec((D0, D_PROJ), lambda i: (0, 0)),
            pl.BlockSpec((D1, D_PROJ), lambda i: (0, 0)),
            pl.BlockSpec((D2, D_PROJ), lambda i: (0, 0)),
        ],
        out_specs=pl.BlockSpec((BT, D_PROJ), lambda i: (i, 0)),
        out_shape=jax.ShapeDtypeStruct((N_TOKENS, D_PROJ), jnp.float32),
    )(g0, g1, g2, p0t, p1t, p2t)


def kernel(input, emb_0, emb_1, emb_2, proj_0, proj_1, proj_2):
    idx_flat = input.T.reshape(-1).astype(jnp.int32)
    e0 = jnp.concatenate(
        [emb_0, jnp.zeros((ZPAD, D0), jnp.float32)], axis=0)
    e1 = jnp.concatenate(
        [emb_1.reshape(-1), jnp.zeros(ZPAD * D1, jnp.float32)]
    ).reshape(V1 + ZPAD, D1)
    e2 = jnp.concatenate(
        [emb_2.reshape(-1), jnp.zeros(ZPAD * D2, jnp.float32)]
    ).reshape(V2 + ZPAD, D2)
    g0, g1 = _sc_gather01_kernel()(idx_flat, e0, e1)
    (g2,) = _sc_gather2_kernel()(idx_flat, e2)
    out = _tc_project(g0, g1, g2, proj_0.T, proj_1.T, proj_2.T)
    return out.reshape(50, 4096, D_PROJ).transpose(1, 0, 2)


# speedup vs baseline: 13.7063x; 1.0468x over previous
import functools

import jax
import jax.numpy as jnp
from jax import lax
from jax.experimental import pallas as pl
from jax.experimental.pallas import tpu as pltpu
from jax.experimental.pallas import tpu_sc as plsc

D_PROJ = 128
CUT0 = 20000
CUT1 = 100000
V0, D0 = 20000, 128
V1, D1 = 80000, 32
V2, D2 = 900000, 8

N_TOKENS = 4096 * 50
ZPAD = 2048
NC, NS = 2, 16
NW = NC * NS
TOK_PER_W = N_TOKENS // NW
CHUNK = 128
CHUNKS_PER_W = TOK_PER_W // CHUNK


def _sc_gather01_body(idx_hbm, emb0, emb1, g0_hbm, g1_hbm,
                      idx_v, i0_v, i1_v, g0_v, g1_v, sems):
    wid = lax.axis_index("s") * NC + lax.axis_index("c")

    def fire(ci, slot):
        base = (wid * CHUNKS_PER_W + ci) * CHUNK
        pltpu.sync_copy(idx_hbm.at[pl.ds(base, CHUNK)], idx_v.at[slot])

        def vec_body(j, c):
            v = idx_v[slot, pl.ds(j * 16, 16)]
            z = v & (ZPAD - 1)
            i0_v[slot, pl.ds(j * 16, 16)] = jnp.where(v < CUT0, v, V0 + z)
            i1_v[slot, pl.ds(j * 16, 16)] = jnp.where(
                (v >= CUT0) & (v < CUT1), v - CUT0, V1 + z)
            return c

        lax.fori_loop(0, CHUNK // 16, vec_body, 0)
        pltpu.async_copy(emb0.at[i0_v.at[slot]], g0_v.at[slot],
                         sems.at[slot, 0])
        pltpu.async_copy(emb1.at[i1_v.at[slot]], g1_v.at[slot],
                         sems.at[slot, 1])

    def drain(ci, slot):
        g = wid * CHUNKS_PER_W + ci
        base = g * CHUNK
        b = g // 8
        c = g % 8
        pltpu.make_async_copy(emb0.at[i0_v.at[slot]], g0_v.at[slot],
                              sems.at[slot, 0]).wait()
        pltpu.make_async_copy(emb1.at[i1_v.at[slot]], g1_v.at[slot],
                              sems.at[slot, 1]).wait()
        pltpu.sync_copy(g0_v.at[slot], g0_hbm.at[pl.ds(base, CHUNK)])
        pltpu.sync_copy(
            g1_v.at[slot],
            g1_hbm.at[pl.ds(b * 256 + (c % 2) * CHUNK, CHUNK),
                      pl.ds(32 * (c // 2), 32)])

    fire(0, 0)

    def pair_body(k, carry):
        fire(2 * k + 1, 1)
        drain(2 * k, 0)

        @pl.when(k < CHUNKS_PER_W // 2 - 1)
        def _():
            fire(2 * k + 2, 0)

        drain(2 * k + 1, 1)
        return carry

    lax.fori_loop(0, CHUNKS_PER_W // 2, pair_body, 0)


def _sc_gather2_body(idx_hbm, emb2, g2_hbm, idx_v, i2_v, g2_v, sems):
    wid = lax.axis_index("s") * NC + lax.axis_index("c")

    def fire(ci, slot):
        base = (wid * CHUNKS_PER_W + ci) * CHUNK
        pltpu.sync_copy(idx_hbm.at[pl.ds(base, CHUNK)], idx_v.at[slot])

        def vec_body(j, c):
            v = idx_v[slot, pl.ds(j * 16, 16)]
            z = v & (ZPAD - 1)
            i2_v[slot, pl.ds(j * 16, 16)] = jnp.where(
                v >= CUT1, v - CUT1, V2 + z)
            return c

        lax.fori_loop(0, CHUNK // 16, vec_body, 0)
        pltpu.async_copy(emb2.at[i2_v.at[slot]], g2_v.at[slot],
                         sems.at[slot])

    def drain(ci, slot):
        g = wid * CHUNKS_PER_W + ci
        b = g // 8
        c = g % 8
        pltpu.make_async_copy(emb2.at[i2_v.at[slot]], g2_v.at[slot],
                              sems.at[slot]).wait()
        pltpu.sync_copy(
            g2_v.at[slot, pl.ds(0, 64)],
            g2_hbm.at[pl.ds(b * 64, 64), pl.ds(8 * (2 * c), 8)])
        pltpu.sync_copy(
            g2_v.at[slot, pl.ds(64, 64)],
            g2_hbm.at[pl.ds(b * 64, 64), pl.ds(8 * (2 * c + 1), 8)])

    fire(0, 0)

    def pair_body(k, carry):
        fire(2 * k + 1, 1)
        drain(2 * k, 0)

        @pl.when(k < CHUNKS_PER_W // 2 - 1)
        def _():
            fire(2 * k + 2, 0)

        drain(2 * k + 1, 1)
        return carry

    lax.fori_loop(0, CHUNKS_PER_W // 2, pair_body, 0)


@functools.cache
def _sc_gather01_kernel():
    return functools.partial(
        pl.kernel,
        mesh=plsc.VectorSubcoreMesh(core_axis_name="c", subcore_axis_name="s"),
        compiler_params=pltpu.CompilerParams(use_tc_tiling_on_sc=False),
        out_type=[
            jax.ShapeDtypeStruct((N_TOKENS, D0), jnp.float32),
            jax.ShapeDtypeStruct((N_TOKENS // 4, 128), jnp.float32),
        ],
        scratch_types=[
            pltpu.VMEM((2, CHUNK), jnp.int32),
            pltpu.VMEM((2, CHUNK), jnp.int32),
            pltpu.VMEM((2, CHUNK), jnp.int32),
            pltpu.VMEM((2, CHUNK, D0), jnp.float32),
            pltpu.VMEM((2, CHUNK, D1), jnp.float32),
            pltpu.SemaphoreType.DMA((2, 2)),
        ],
    )(_sc_gather01_body)


@functools.cache
def _sc_gather2_kernel():
    return functools.partial(
        pl.kernel,
        mesh=plsc.VectorSubcoreMesh(core_axis_name="c", subcore_axis_name="s"),
        compiler_params=pltpu.CompilerParams(use_tc_tiling_on_sc=False),
        out_type=[
            jax.ShapeDtypeStruct((N_TOKENS // 16, 128), jnp.float32),
        ],
        scratch_types=[
            pltpu.VMEM((2, CHUNK), jnp.int32),
            pltpu.VMEM((2, CHUNK), jnp.int32),
            pltpu.VMEM((2, CHUNK, D2), jnp.float32),
            pltpu.SemaphoreType.DMA((2,)),
        ],
    )(_sc_gather2_body)


BT = 4096
N_BLOCKS = N_TOKENS // BT


def _tc_project_body(g0_ref, g1_ref, g2_ref, p0_ref, p1_ref, p2_ref, out_ref):
    acc = jnp.dot(g0_ref[...], p0_ref[...],
                  preferred_element_type=jnp.float32)
    g1p = g1_ref[...]
    acc += jnp.concatenate(
        [jnp.dot(g1p[256 * u:256 * u + 256, 32 * q:32 * q + 32], p1_ref[...],
                 preferred_element_type=jnp.float32)
         for u in range(BT // 1024) for q in range(4)], axis=0)
    g2p = g2_ref[...]
    acc += jnp.concatenate(
        [jnp.dot(g2p[64 * u:64 * u + 64, 8 * q:8 * q + 8], p2_ref[...],
                 preferred_element_type=jnp.float32)
         for u in range(BT // 1024) for q in range(16)], axis=0)
    out_ref[...] = acc


def _tc_project(g0, g1, g2, p0t, p1t, p2t):
    return pl.pallas_call(
        _tc_project_body,
        grid=(N_BLOCKS,),
        in_specs=[
            pl.BlockSpec((BT, D0), lambda i: (i, 0)),
            pl.BlockSpec((BT // 4, 128), lambda i: (i, 0)),
            pl.BlockSpec((BT // 16, 128), lambda i: (i, 0)),
            pl.BlockSpec((D0, D_PROJ), lambda i: (0, 0)),
            pl.BlockSpec((D1, D_PROJ), lambda i: (0, 0)),
            pl.BlockSpec((D2, D_PROJ), lambda i: (0, 0)),
        ],
        out_specs=pl.BlockSpec((BT, D_PROJ), lambda i: (i, 0)),
        out_shape=jax.ShapeDtypeStruct((N_TOKENS, D_PROJ), jnp.float32),
    )(g0, g1, g2, p0t, p1t, p2t)


def kernel(input, emb_0, emb_1, emb_2, proj_0, proj_1, proj_2):
    idx_flat = input.T.reshape(-1).astype(jnp.int32)
    e0 = jnp.concatenate(
        [emb_0, jnp.zeros((ZPAD, D0), jnp.float32)], axis=0)
    e1 = jnp.concatenate(
        [emb_1.reshape(-1), jnp.zeros(ZPAD * D1, jnp.float32)]
    ).reshape(V1 + ZPAD, D1)
    e2 = jnp.concatenate(
        [emb_2.reshape(-1), jnp.zeros(ZPAD * D2, jnp.float32)]
    ).reshape(V2 + ZPAD, D2)
    g0, g1 = _sc_gather01_kernel()(idx_flat, e0, e1)
    (g2,) = _sc_gather2_kernel()(idx_flat, e2)
    out = _tc_project(g0, g1, g2, proj_0.T, proj_1.T, proj_2.T)
    return out.reshape(50, 4096, D_PROJ).transpose(1, 0, 2)


# speedup vs baseline: 14.0641x; 1.0261x over previous
import functools

import jax
import jax.numpy as jnp
from jax import lax
from jax.experimental import pallas as pl
from jax.experimental.pallas import tpu as pltpu
from jax.experimental.pallas import tpu_sc as plsc

D_PROJ = 128
CUT0 = 20000
CUT1 = 100000
V0, D0 = 20000, 128
V1, D1 = 80000, 32
V2, D2 = 900000, 8

N_TOKENS = 4096 * 50
ZPAD = 2048
NC, NS = 2, 16
NW = NC * NS
TOK_PER_W = N_TOKENS // NW
CHUNK = 128
CHUNKS_PER_W = TOK_PER_W // CHUNK


def _sc_gather01_body(idx_hbm, emb0, emb1, g0_hbm, g1_hbm,
                      idx_v, i0_v, i1_v, g0_v, g1_v, sems):
    wid = lax.axis_index("s") * NC + lax.axis_index("c")

    def fire(ci, slot):
        base = (wid * CHUNKS_PER_W + ci) * CHUNK
        pltpu.sync_copy(idx_hbm.at[pl.ds(base, CHUNK)], idx_v.at[slot])

        def vec_body(j, c):
            v = idx_v[slot, pl.ds(j * 16, 16)]
            z = v & (ZPAD - 1)
            i0_v[slot, pl.ds(j * 16, 16)] = jnp.where(v < CUT0, v, V0 + z)
            i1_v[slot, pl.ds(j * 16, 16)] = jnp.where(
                (v >= CUT0) & (v < CUT1), v - CUT0, V1 + z)
            return c

        lax.fori_loop(0, CHUNK // 16, vec_body, 0)
        pltpu.async_copy(emb0.at[i0_v.at[slot]], g0_v.at[slot],
                         sems.at[slot, 0])
        pltpu.async_copy(emb1.at[i1_v.at[slot]], g1_v.at[slot],
                         sems.at[slot, 1])

    def drain(ci, slot):
        g = wid * CHUNKS_PER_W + ci
        base = g * CHUNK
        b = g // 8
        c = g % 8
        pltpu.make_async_copy(emb0.at[i0_v.at[slot]], g0_v.at[slot],
                              sems.at[slot, 0]).wait()
        pltpu.make_async_copy(emb1.at[i1_v.at[slot]], g1_v.at[slot],
                              sems.at[slot, 1]).wait()
        pltpu.sync_copy(g0_v.at[slot], g0_hbm.at[pl.ds(base, CHUNK)])
        pltpu.sync_copy(
            g1_v.at[slot],
            g1_hbm.at[pl.ds(b * 256 + (c % 2) * CHUNK, CHUNK),
                      pl.ds(32 * (c // 2), 32)])

    fire(0, 0)

    def pair_body(k, carry):
        fire(2 * k + 1, 1)
        drain(2 * k, 0)

        @pl.when(k < CHUNKS_PER_W // 2 - 1)
        def _():
            fire(2 * k + 2, 0)

        drain(2 * k + 1, 1)
        return carry

    lax.fori_loop(0, CHUNKS_PER_W // 2, pair_body, 0)


def _sc_gather2_body(idx_hbm, emb2, g2_hbm, idx_v, i2_v, g2_v, sems):
    wid = lax.axis_index("s") * NC + lax.axis_index("c")

    def fire(ci, slot):
        base = (wid * CHUNKS_PER_W + ci) * CHUNK
        pltpu.sync_copy(idx_hbm.at[pl.ds(base, CHUNK)], idx_v.at[slot])

        def vec_body(j, c):
            v = idx_v[slot, pl.ds(j * 16, 16)]
            z = v & (ZPAD - 1)
            i2_v[slot, pl.ds(j * 16, 16)] = jnp.where(
                v >= CUT1, v - CUT1, V2 + z)
            return c

        lax.fori_loop(0, CHUNK // 16, vec_body, 0)
        pltpu.async_copy(emb2.at[i2_v.at[slot]], g2_v.at[slot],
                         sems.at[slot])

    def drain(ci, slot):
        g = wid * CHUNKS_PER_W + ci
        b = g // 8
        c = g % 8
        pltpu.make_async_copy(emb2.at[i2_v.at[slot]], g2_v.at[slot],
                              sems.at[slot]).wait()
        pltpu.sync_copy(
            g2_v.at[slot, pl.ds(0, 64)],
            g2_hbm.at[pl.ds(b * 64, 64), pl.ds(8 * (2 * c), 8)])
        pltpu.sync_copy(
            g2_v.at[slot, pl.ds(64, 64)],
            g2_hbm.at[pl.ds(b * 64, 64), pl.ds(8 * (2 * c + 1), 8)])

    fire(0, 0)

    def pair_body(k, carry):
        fire(2 * k + 1, 1)
        drain(2 * k, 0)

        @pl.when(k < CHUNKS_PER_W // 2 - 1)
        def _():
            fire(2 * k + 2, 0)

        drain(2 * k + 1, 1)
        return carry

    lax.fori_loop(0, CHUNKS_PER_W // 2, pair_body, 0)


@functools.cache
def _sc_gather01_kernel():
    return functools.partial(
        pl.kernel,
        mesh=plsc.VectorSubcoreMesh(core_axis_name="c", subcore_axis_name="s"),
        compiler_params=pltpu.CompilerParams(use_tc_tiling_on_sc=False),
        out_type=[
            jax.ShapeDtypeStruct((N_TOKENS, D0), jnp.float32),
            jax.ShapeDtypeStruct((N_TOKENS // 4, 128), jnp.float32),
        ],
        scratch_types=[
            pltpu.VMEM((2, CHUNK), jnp.int32),
            pltpu.VMEM((2, CHUNK), jnp.int32),
            pltpu.VMEM((2, CHUNK), jnp.int32),
            pltpu.VMEM((2, CHUNK, D0), jnp.float32),
            pltpu.VMEM((2, CHUNK, D1), jnp.float32),
            pltpu.SemaphoreType.DMA((2, 2)),
        ],
    )(_sc_gather01_body)


@functools.cache
def _sc_gather2_kernel():
    return functools.partial(
        pl.kernel,
        mesh=plsc.VectorSubcoreMesh(core_axis_name="c", subcore_axis_name="s"),
        compiler_params=pltpu.CompilerParams(use_tc_tiling_on_sc=False),
        out_type=[
            jax.ShapeDtypeStruct((N_TOKENS // 16, 128), jnp.float32),
        ],
        scratch_types=[
            pltpu.VMEM((2, CHUNK), jnp.int32),
            pltpu.VMEM((2, CHUNK), jnp.int32),
            pltpu.VMEM((2, CHUNK, D2), jnp.float32),
            pltpu.SemaphoreType.DMA((2,)),
        ],
    )(_sc_gather2_body)


BT = 8192
N_BLOCKS = N_TOKENS // BT


def _tc_project_body(g0_ref, g1_ref, g2_ref, p0_ref, p1_ref, p2_ref, out_ref):
    acc = jnp.dot(g0_ref[...], p0_ref[...],
                  preferred_element_type=jnp.float32)
    g1p = g1_ref[...]
    acc += jnp.concatenate(
        [jnp.dot(g1p[256 * u:256 * u + 256, 32 * q:32 * q + 32], p1_ref[...],
                 preferred_element_type=jnp.float32)
         for u in range(BT // 1024) for q in range(4)], axis=0)
    g2p = g2_ref[...]
    acc += jnp.concatenate(
        [jnp.dot(g2p[64 * u:64 * u + 64, 8 * q:8 * q + 8], p2_ref[...],
                 preferred_element_type=jnp.float32)
         for u in range(BT // 1024) for q in range(16)], axis=0)
    out_ref[...] = acc


def _tc_project(g0, g1, g2, p0t, p1t, p2t):
    return pl.pallas_call(
        _tc_project_body,
        grid=(N_BLOCKS,),
        in_specs=[
            pl.BlockSpec((BT, D0), lambda i: (i, 0)),
            pl.BlockSpec((BT // 4, 128), lambda i: (i, 0)),
            pl.BlockSpec((BT // 16, 128), lambda i: (i, 0)),
            pl.BlockSpec((D0, D_PROJ), lambda i: (0, 0)),
            pl.BlockSpec((D1, D_PROJ), lambda i: (0, 0)),
            pl.BlockSpec((D2, D_PROJ), lambda i: (0, 0)),
        ],
        out_specs=pl.BlockSpec((BT, D_PROJ), lambda i: (i, 0)),
        out_shape=jax.ShapeDtypeStruct((N_TOKENS, D_PROJ), jnp.float32),
    )(g0, g1, g2, p0t, p1t, p2t)


def kernel(input, emb_0, emb_1, emb_2, proj_0, proj_1, proj_2):
    idx_flat = input.T.reshape(-1).astype(jnp.int32)
    e0 = jnp.concatenate(
        [emb_0, jnp.zeros((ZPAD, D0), jnp.float32)], axis=0)
    e1 = jnp.concatenate(
        [emb_1.reshape(-1), jnp.zeros(ZPAD * D1, jnp.float32)]
    ).reshape(V1 + ZPAD, D1)
    e2 = jnp.concatenate(
        [emb_2.reshape(-1), jnp.zeros(ZPAD * D2, jnp.float32)]
    ).reshape(V2 + ZPAD, D2)
    g0, g1 = _sc_gather01_kernel()(idx_flat, e0, e1)
    (g2,) = _sc_gather2_kernel()(idx_flat, e2)
    out = _tc_project(g0, g1, g2, proj_0.T, proj_1.T, proj_2.T)
    return out.reshape(50, 4096, D_PROJ).transpose(1, 0, 2)


# speedup vs baseline: 14.2065x; 1.0101x over previous
import functools

import jax
import jax.numpy as jnp
from jax import lax
from jax.experimental import pallas as pl
from jax.experimental.pallas import tpu as pltpu
from jax.experimental.pallas import tpu_sc as plsc

D_PROJ = 128
CUT0 = 20000
CUT1 = 100000
V0, D0 = 20000, 128
V1, D1 = 80000, 32
V2, D2 = 900000, 8

N_TOKENS = 4096 * 50
ZPAD = 2048
NC, NS = 2, 16
NW = NC * NS
TOK_PER_W = N_TOKENS // NW
CHUNK = 128
CHUNKS_PER_W = TOK_PER_W // CHUNK


def _sc_gather01_body(idx_hbm, emb0, emb1, g0_hbm, g1_hbm,
                      idx_v, i0_v, i1_v, g0_v, g1_v, sems):
    wid = lax.axis_index("s") * NC + lax.axis_index("c")

    def fire(ci, slot):
        base = (wid * CHUNKS_PER_W + ci) * CHUNK
        pltpu.sync_copy(idx_hbm.at[pl.ds(base, CHUNK)], idx_v.at[slot])

        def vec_body(j, c):
            v = idx_v[slot, pl.ds(j * 16, 16)]
            z = v & (ZPAD - 1)
            i0_v[slot, pl.ds(j * 16, 16)] = jnp.where(v < CUT0, v, V0 + z)
            i1_v[slot, pl.ds(j * 16, 16)] = jnp.where(
                (v >= CUT0) & (v < CUT1), v - CUT0, V1 + z)
            return c

        lax.fori_loop(0, CHUNK // 16, vec_body, 0)
        pltpu.async_copy(emb0.at[i0_v.at[slot]], g0_v.at[slot],
                         sems.at[slot, 0])
        pltpu.async_copy(emb1.at[i1_v.at[slot]], g1_v.at[slot],
                         sems.at[slot, 1])

    def drain(ci, slot):
        g = wid * CHUNKS_PER_W + ci
        base = g * CHUNK
        b = g // 8
        c = g % 8
        pltpu.make_async_copy(emb0.at[i0_v.at[slot]], g0_v.at[slot],
                              sems.at[slot, 0]).wait()
        pltpu.make_async_copy(emb1.at[i1_v.at[slot]], g1_v.at[slot],
                              sems.at[slot, 1]).wait()
        pltpu.sync_copy(g0_v.at[slot], g0_hbm.at[pl.ds(base, CHUNK)])
        pltpu.sync_copy(
            g1_v.at[slot],
            g1_hbm.at[pl.ds(b * 256 + (c % 2) * CHUNK, CHUNK),
                      pl.ds(32 * (c // 2), 32)])

    fire(0, 0)

    def pair_body(k, carry):
        fire(2 * k + 1, 1)
        drain(2 * k, 0)

        @pl.when(k < CHUNKS_PER_W // 2 - 1)
        def _():
            fire(2 * k + 2, 0)

        drain(2 * k + 1, 1)
        return carry

    lax.fori_loop(0, CHUNKS_PER_W // 2, pair_body, 0)


def _sc_gather2_body(idx_hbm, emb2, g2_hbm, idx_v, i2_v, g2_v, sems):
    wid = lax.axis_index("s") * NC + lax.axis_index("c")

    def fire(ci, slot):
        base = (wid * CHUNKS_PER_W + ci) * CHUNK
        pltpu.sync_copy(idx_hbm.at[pl.ds(base, CHUNK)], idx_v.at[slot])

        def vec_body(j, c):
            v = idx_v[slot, pl.ds(j * 16, 16)]
            z = v & (ZPAD - 1)
            i2_v[slot, pl.ds(j * 16, 16)] = jnp.where(
                v >= CUT1, v - CUT1, V2 + z)
            return c

        lax.fori_loop(0, CHUNK // 16, vec_body, 0)
        pltpu.async_copy(emb2.at[i2_v.at[slot]], g2_v.at[slot],
                         sems.at[slot])

    def drain(ci, slot):
        g = wid * CHUNKS_PER_W + ci
        b = g // 8
        c = g % 8
        pltpu.make_async_copy(emb2.at[i2_v.at[slot]], g2_v.at[slot],
                              sems.at[slot]).wait()
        pltpu.sync_copy(
            g2_v.at[slot, pl.ds(0, 64)],
            g2_hbm.at[pl.ds(b * 64, 64), pl.ds(8 * (2 * c), 8)])
        pltpu.sync_copy(
            g2_v.at[slot, pl.ds(64, 64)],
            g2_hbm.at[pl.ds(b * 64, 64), pl.ds(8 * (2 * c + 1), 8)])

    fire(0, 0)

    def pair_body(k, carry):
        fire(2 * k + 1, 1)
        drain(2 * k, 0)

        @pl.when(k < CHUNKS_PER_W // 2 - 1)
        def _():
            fire(2 * k + 2, 0)

        drain(2 * k + 1, 1)
        return carry

    lax.fori_loop(0, CHUNKS_PER_W // 2, pair_body, 0)


@functools.cache
def _sc_gather01_kernel():
    return functools.partial(
        pl.kernel,
        mesh=plsc.VectorSubcoreMesh(core_axis_name="c", subcore_axis_name="s"),
        compiler_params=pltpu.CompilerParams(use_tc_tiling_on_sc=False),
        out_type=[
            jax.ShapeDtypeStruct((N_TOKENS, D0), jnp.float32),
            jax.ShapeDtypeStruct((N_TOKENS // 4, 128), jnp.float32),
        ],
        scratch_types=[
            pltpu.VMEM((2, CHUNK), jnp.int32),
            pltpu.VMEM((2, CHUNK), jnp.int32),
            pltpu.VMEM((2, CHUNK), jnp.int32),
            pltpu.VMEM((2, CHUNK, D0), jnp.float32),
            pltpu.VMEM((2, CHUNK, D1), jnp.float32),
            pltpu.SemaphoreType.DMA((2, 2)),
        ],
    )(_sc_gather01_body)


@functools.cache
def _sc_gather2_kernel():
    return functools.partial(
        pl.kernel,
        mesh=plsc.VectorSubcoreMesh(core_axis_name="c", subcore_axis_name="s"),
        compiler_params=pltpu.CompilerParams(use_tc_tiling_on_sc=False),
        out_type=[
            jax.ShapeDtypeStruct((N_TOKENS // 16, 128), jnp.float32),
        ],
        scratch_types=[
            pltpu.VMEM((2, CHUNK), jnp.int32),
            pltpu.VMEM((2, CHUNK), jnp.int32),
            pltpu.VMEM((2, CHUNK, D2), jnp.float32),
            pltpu.SemaphoreType.DMA((2,)),
        ],
    )(_sc_gather2_body)


BT = 16384
N_BLOCKS = N_TOKENS // BT


def _tc_project_body(g0_ref, g1_ref, g2_ref, p0_ref, p1_ref, p2_ref, out_ref):
    acc = jnp.dot(g0_ref[...], p0_ref[...],
                  preferred_element_type=jnp.float32)
    g1p = g1_ref[...]
    acc += jnp.concatenate(
        [jnp.dot(g1p[256 * u:256 * u + 256, 32 * q:32 * q + 32], p1_ref[...],
                 preferred_element_type=jnp.float32)
         for u in range(BT // 1024) for q in range(4)], axis=0)
    g2p = g2_ref[...]
    acc += jnp.concatenate(
        [jnp.dot(g2p[64 * u:64 * u + 64, 8 * q:8 * q + 8], p2_ref[...],
                 preferred_element_type=jnp.float32)
         for u in range(BT // 1024) for q in range(16)], axis=0)
    out_ref[...] = acc


def _tc_project(g0, g1, g2, p0t, p1t, p2t):
    return pl.pallas_call(
        _tc_project_body,
        grid=(N_BLOCKS,),
        in_specs=[
            pl.BlockSpec((BT, D0), lambda i: (i, 0)),
            pl.BlockSpec((BT // 4, 128), lambda i: (i, 0)),
            pl.BlockSpec((BT // 16, 128), lambda i: (i, 0)),
            pl.BlockSpec((D0, D_PROJ), lambda i: (0, 0)),
            pl.BlockSpec((D1, D_PROJ), lambda i: (0, 0)),
            pl.BlockSpec((D2, D_PROJ), lambda i: (0, 0)),
        ],
        out_specs=pl.BlockSpec((BT, D_PROJ), lambda i: (i, 0)),
        out_shape=jax.ShapeDtypeStruct((N_TOKENS, D_PROJ), jnp.float32),
    )(g0, g1, g2, p0t, p1t, p2t)


def kernel(input, emb_0, emb_1, emb_2, proj_0, proj_1, proj_2):
    idx_flat = input.T.reshape(-1).astype(jnp.int32)
    e0 = jnp.concatenate(
        [emb_0, jnp.zeros((ZPAD, D0), jnp.float32)], axis=0)
    e1 = jnp.concatenate(
        [emb_1.reshape(-1), jnp.zeros(ZPAD * D1, jnp.float32)]
    ).reshape(V1 + ZPAD, D1)
    e2 = jnp.concatenate(
        [emb_2.reshape(-1), jnp.zeros(ZPAD * D2, jnp.float32)]
    ).reshape(V2 + ZPAD, D2)
    g0, g1 = _sc_gather01_kernel()(idx_flat, e0, e1)
    (g2,) = _sc_gather2_kernel()(idx_flat, e2)
    out = _tc_project(g0, g1, g2, proj_0.T, proj_1.T, proj_2.T)
    return out.reshape(50, 4096, D_PROJ).transpose(1, 0, 2)
